# baseline (speedup 1.0000x reference)
import functools

import jax
import jax.numpy as jnp
from jax import lax
from jax.experimental import pallas as pl
from jax.experimental.pallas import tpu as pltpu
from jax.experimental.pallas import tpu_sc as plsc

N_TASK = 5000
N_AG = 5000
D = 128
DEG = 64
E = N_AG * DEG
FIN_TASK_TYPE = 3
EPS = 1e-5
NEG_SLOPE = 0.01

NC, NS = 2, 16
NW = NC * NS
FPW = D // NW
CH = 1600
NCH = E // CH
AGC = CH // DEG
EFW = E // NW

_SC_MESH = plsc.VectorSubcoreMesh(
    core_axis_name="c", subcore_axis_name="s", num_cores=NC, num_subcores=NS)
_SC_PARAMS = pltpu.CompilerParams(needs_layout_passes=False)


def _proj_body(nt_ref, na_ref, ws_ref, wd_ref, t_ref, a_ref):
    t_ref[...] = jnp.dot(nt_ref[...], ws_ref[...],
                         preferred_element_type=jnp.float32)
    a_ref[...] = jnp.dot(na_ref[...], wd_ref[...],
                         preferred_element_type=jnp.float32)


def _proj(nf_t, nf_a, w1s, w1d):
    blk = 1000
    return pl.pallas_call(
        _proj_body,
        grid=(N_TASK // blk,),
        in_specs=[
            pl.BlockSpec((blk, D), lambda i: (i, 0)),
            pl.BlockSpec((blk, D), lambda i: (i, 0)),
            pl.BlockSpec((D, D), lambda i: (0, 0)),
            pl.BlockSpec((D, D), lambda i: (0, 0)),
        ],
        out_specs=[
            pl.BlockSpec((blk, D), lambda i: (i, 0)),
            pl.BlockSpec((blk, D), lambda i: (i, 0)),
        ],
        out_shape=[
            jax.ShapeDtypeStruct((N_TASK, D), jnp.float32),
            jax.ShapeDtypeStruct((N_AG, D), jnp.float32),
        ],
    )(nf_t, nf_a, w1s, w1d)


def _load_tables(tpT, apT, wid, tl, al):
    for c in range(FPW):
        pltpu.sync_copy(tpT.at[pl.ds((wid * FPW + c) * N_TASK, N_TASK)],
                        tl[c])
    pltpu.sync_copy(apT.at[pl.ds(wid * FPW * N_AG, FPW * N_AG)], al)


def _s1_body(tpT, apT, src, part, t0, t1, t2, t3, al_v, sb0, sb1, acc_v,
             sem0, sem1):
    wid = lax.axis_index("s") * NC + lax.axis_index("c")
    tl = [t0, t1, t2, t3]
    _load_tables(tpT, apT, wid, tl, al_v)

    def start_src(ch, sb, sem):
        pltpu.async_copy(src.at[pl.ds(ch * CH, CH)], sb, sem)

    def wait_src(ch, sb, sem):
        pltpu.make_async_copy(src.at[pl.ds(ch * CH, CH)], sb, sem).wait()

    zero = jnp.zeros((16,), jnp.float32)

    def process(ch, sb, accs):
        def agent_body(a, accs):
            s, q = accs
            agv = jnp.broadcast_to(ch * AGC + a, (16,)).astype(jnp.int32)
            asp = [plsc.load_gather(al_v, [agv + c * N_AG])
                   for c in range(FPW)]
            s = list(s)
            q = list(q)
            for g in range(DEG // 16):
                srcv = sb[pl.ds(a * DEG + g * 16, 16)]
                for c in range(FPW):
                    tg = plsc.load_gather(tl[c], [srcv])
                    h = tg + asp[c]
                    s[c] = s[c] + h
                    q[c] = q[c] + h * h
            return (tuple(s), tuple(q))

        return lax.fori_loop(0, AGC, agent_body, accs)

    init = (tuple(zero for _ in range(FPW)), tuple(zero for _ in range(FPW)))

    def body(p, accs):
        ch0, ch1 = 2 * p, 2 * p + 1
        pltpu.sync_copy(src.at[pl.ds(ch0 * CH, CH)], sb0)
        accs = process(ch0, sb0, accs)
        pltpu.sync_copy(src.at[pl.ds(ch1 * CH, CH)], sb1)
        return process(ch1, sb1, accs)

    fs, fq = lax.fori_loop(0, NCH // 2, body, init)
    for c in range(FPW):
        acc_v[pl.ds(c * 16, 16)] = fs[c]
        acc_v[pl.ds((FPW + c) * 16, 16)] = fq[c]
    pltpu.sync_copy(acc_v, part.at[wid])


@functools.partial(
    pl.kernel,
    out_type=jax.ShapeDtypeStruct((NW, 2 * FPW * 16), jnp.float32),
    mesh=_SC_MESH,
    compiler_params=_SC_PARAMS,
    scratch_types=[
        pltpu.VMEM((N_TASK,), jnp.float32),
        pltpu.VMEM((N_TASK,), jnp.float32),
        pltpu.VMEM((N_TASK,), jnp.float32),
        pltpu.VMEM((N_TASK,), jnp.float32),
        pltpu.VMEM((FPW * N_AG,), jnp.float32),
        pltpu.VMEM((CH,), jnp.int32),
        pltpu.VMEM((CH,), jnp.int32),
        pltpu.VMEM((2 * FPW * 16,), jnp.float32),
        pltpu.SemaphoreType.DMA,
        pltpu.SemaphoreType.DMA,
    ],
)
def _s1(tpT, apT, src, part, t0, t1, t2, t3, al_v, sb0, sb1, acc_v,
        sem0, sem1):
    _s1_body(tpT, apT, src, part, t0, t1, t2, t3, al_v, sb0, sb1, acc_v,
             sem0, sem1)


_AST = 5008


def _s2_body(tpT, apT, src, kbw, fin, ps, fing,
             t0, t1, t2, t3, al_v, sb0, sb1, pb0, pb1, kbw_v, fint_v,
             fsrc_v, finb_v, sem0, sem1, psem0, psem1, fsem):
    wid = lax.axis_index("s") * NC + lax.axis_index("c")
    tl = [t0, t1, t2, t3]
    for c in range(FPW):
        toff = pl.multiple_of((wid * FPW + c) * N_TASK, 8)
        pltpu.sync_copy(tpT.at[pl.ds(toff, N_TASK)],
                        tl[c].at[pl.ds(0, N_TASK)])
        aoff = pl.multiple_of((wid * FPW + c) * N_AG, 8)
        pltpu.sync_copy(apT.at[pl.ds(aoff, N_AG)],
                        al_v.at[pl.ds(c * _AST, N_AG)])
    pltpu.sync_copy(kbw, kbw_v)
    pltpu.sync_copy(fin, fint_v)

    widv = jnp.broadcast_to(wid * FPW, (16,)).astype(jnp.int32)
    k1sp = [plsc.load_gather(kbw_v, [widv + c]) for c in range(FPW)]
    b1sp = [plsc.load_gather(kbw_v, [widv + (D + c)]) for c in range(FPW)]
    w2sp = [plsc.load_gather(kbw_v, [widv + (2 * D + c)])
            for c in range(FPW)]

    def scale_t(i, cy):
        for c in range(FPW):
            tl[c][pl.ds(i * 16, 16)] = tl[c][pl.ds(i * 16, 16)] * k1sp[c]
        return cy

    lax.fori_loop(0, _AST // 16, scale_t, jnp.int32(0))

    def scale_a(i, cy):
        for c in range(FPW):
            al_v[pl.ds(c * _AST + i * 16, 16)] = (
                al_v[pl.ds(c * _AST + i * 16, 16)] * k1sp[c] + b1sp[c])
        return cy

    lax.fori_loop(0, _AST // 16, scale_a, jnp.int32(0))

    def start_src(ch, sb, sem):
        pltpu.async_copy(src.at[pl.ds(ch * CH, CH)], sb, sem)

    def wait_src(ch, sb, sem):
        pltpu.make_async_copy(src.at[pl.ds(ch * CH, CH)], sb, sem).wait()

    def start_ps(ch, pb, psem):
        off = pl.multiple_of(wid * E + ch * CH, 8)
        pltpu.async_copy(pb, ps.at[pl.ds(off, CH)], psem)

    def wait_ps(ch, pb, psem):
        off = pl.multiple_of(wid * E + ch * CH, 8)
        pltpu.make_async_copy(pb, ps.at[pl.ds(off, CH)], psem).wait()

    def process(ch, sb, pb):
        def agent_body(a):
            agv = jnp.broadcast_to(ch * AGC + a, (16,)).astype(jnp.int32)
            srcvs = [sb[pl.ds(a * DEG + g * 16, 16)]
                     for g in range(DEG // 16)]
            tgs = [[plsc.load_gather(tl[c], [srcvs[g]]) for c in range(FPW)]
                   for g in range(DEG // 16)]
            ca = [plsc.load_gather(al_v, [agv + c * _AST])
                  for c in range(FPW)]
            for g in range(DEG // 16):
                acc0 = None
                acc1 = None
                for c in range(FPW):
                    hn = tgs[g][c] + ca[c]
                    lr = jnp.maximum(hn, hn * NEG_SLOPE)
                    t = lr * w2sp[c]
                    if c % 2 == 0:
                        acc0 = t if acc0 is None else acc0 + t
                    else:
                        acc1 = t if acc1 is None else acc1 + t
                pb[pl.ds(a * DEG + g * 16, 16)] = acc0 + acc1
            return jnp.int32(0)

        lax.fori_loop(0, AGC, lambda a, cy: agent_body(a), jnp.int32(0))

    start_src(0, sb0, sem0)

    def body(p, carry):
        ch0, ch1 = 2 * p, 2 * p + 1
        pltpu.sync_copy(src.at[pl.ds(ch0 * CH, CH)], sb0)

        @pl.when(p > 0)
        def _():
            wait_ps(ch0 - 2, pb0, psem0)

        process(ch0, sb0, pb0)
        start_ps(ch0, pb0, psem0)

        pltpu.sync_copy(src.at[pl.ds(ch1 * CH, CH)], sb1)

        @pl.when(p > 0)
        def _():
            wait_ps(ch1 - 2, pb1, psem1)

        process(ch1, sb1, pb1)
        start_ps(ch1, pb1, psem1)
        return carry

    lax.fori_loop(0, NCH // 2, body, jnp.int32(0))

    for t in range(5):
        off = pl.multiple_of(wid * EFW + t * 2000, 8)
        pltpu.sync_copy(src.at[pl.ds(off, 2000)],
                        fsrc_v.at[pl.ds(t * 2000, 2000)])

    def fin_body(g, cy):
        srcv = fsrc_v[pl.ds(g * 16, 16)]
        finb_v[pl.ds(g * 16, 16)] = plsc.load_gather(fint_v, [srcv])
        return cy

    lax.fori_loop(0, EFW // 16, fin_body, jnp.int32(0))
    pltpu.sync_copy(finb_v, fing.at[pl.ds(wid * EFW, EFW)])

    wait_ps(NCH - 2, pb0, psem0)
    wait_ps(NCH - 1, pb1, psem1)


@functools.partial(
    pl.kernel,
    out_type=(
        jax.ShapeDtypeStruct((NW * E,), jnp.float32),
        jax.ShapeDtypeStruct((E,), jnp.float32),
    ),
    mesh=_SC_MESH,
    compiler_params=_SC_PARAMS,
    scratch_types=[
        pltpu.VMEM((_AST,), jnp.float32),
        pltpu.VMEM((_AST,), jnp.float32),
        pltpu.VMEM((_AST,), jnp.float32),
        pltpu.VMEM((_AST,), jnp.float32),
        pltpu.VMEM((FPW * _AST,), jnp.float32),
        pltpu.VMEM((CH,), jnp.int32),
        pltpu.VMEM((CH,), jnp.int32),
        pltpu.VMEM((CH,), jnp.float32),
        pltpu.VMEM((CH,), jnp.float32),
        pltpu.VMEM((3 * D,), jnp.float32),
        pltpu.VMEM((N_TASK,), jnp.float32),
        pltpu.VMEM((EFW,), jnp.int32),
        pltpu.VMEM((EFW,), jnp.float32),
        pltpu.SemaphoreType.DMA,
        pltpu.SemaphoreType.DMA,
        pltpu.SemaphoreType.DMA,
        pltpu.SemaphoreType.DMA,
        pltpu.SemaphoreType.DMA,
    ],
)
def _s2(tpT, apT, src, kbw, fin, ps, fing,
        t0, t1, t2, t3, al_v, sb0, sb1, pb0, pb1, kbw_v, fint_v, fsrc_v,
        finb_v, sem0, sem1, psem0, psem1, fsem):
    _s2_body(tpT, apT, src, kbw, fin, ps, fing,
             t0, t1, t2, t3, al_v, sb0, sb1, pb0, pb1, kbw_v, fint_v,
             fsrc_v, finb_v, sem0, sem1, psem0, psem1, fsem)


_TB = 8


def _red_body(ps_ref, s_ref, stat_ref):
    i = pl.program_id(0)
    sblk = jnp.sum(ps_ref[...], axis=0)

    @pl.when(i == 0)
    def _():
        s_ref[...] = sblk

    @pl.when(i > 0)
    def _():
        s_ref[...] += sblk

    @pl.when(i == pl.num_programs(0) - 1)
    def _():
        s_all = s_ref[...]
        stat_ref[0] = jnp.sum(s_all)
        stat_ref[1] = jnp.sum(s_all * s_all)


def _reduce(ps3):
    rows = E // D
    return pl.pallas_call(
        _red_body,
        grid=(NW // _TB,),
        in_specs=[pl.BlockSpec((_TB, rows, D), lambda i: (i, 0, 0))],
        out_specs=[
            pl.BlockSpec((rows, D), lambda i: (0, 0)),
            pl.BlockSpec(memory_space=pltpu.SMEM),
        ],
        out_shape=[
            jax.ShapeDtypeStruct((rows, D), jnp.float32),
            jax.ShapeDtypeStruct((2,), jnp.float32),
        ],
    )(ps3)


def _fin_body(s_ref, f_ref, scal_ref, o_ref):
    k2 = scal_ref[0]
    b2 = scal_ref[1]
    o_ref[...] = jnp.where(f_ref[...] > 0.5, -jnp.inf,
                           s_ref[...] * k2 + b2)


def _final(s2, f2, scal):
    rows = E // D
    return pl.pallas_call(
        _fin_body,
        grid=(1,),
        in_specs=[
            pl.BlockSpec((rows, D), lambda i: (0, 0)),
            pl.BlockSpec((rows, D), lambda i: (0, 0)),
            pl.BlockSpec(memory_space=pltpu.SMEM),
        ],
        out_specs=pl.BlockSpec((rows, D), lambda i: (0, 0)),
        out_shape=jax.ShapeDtypeStruct((rows, D), jnp.float32),
    )(s2, f2, scal)


def kernel(nf, edge_index, node_type, W1, gamma1, beta1, W2, gamma2, beta2):
    src = edge_index[0].astype(jnp.int32)
    nf_t = nf[:N_TASK]
    nf_a = nf[N_TASK:]
    w1s = W1[:D]
    w1d = W1[D:]

    t_proj, a_proj = _proj(nf_t, nf_a, w1s, w1d)
    tpT = t_proj.T.reshape(-1)
    apT = a_proj.T.reshape(-1)

    part = _s1(tpT, apT, src)
    plane = part.reshape(NW, 2 * FPW, 16).sum(axis=-1)
    sums = plane[:, :FPW].reshape(D)
    sqs = plane[:, FPW:].reshape(D)
    mu1 = sums / E
    var1 = sqs / E - mu1 * mu1
    k1 = gamma1 / jnp.sqrt(var1 + EPS)
    b1 = beta1 - mu1 * k1

    kbw = jnp.concatenate([k1, b1, W2[:, 0]])
    fin = (node_type[:N_TASK] == FIN_TASK_TYPE).astype(jnp.float32)

    ps, fing = _s2(tpT, apT, src, kbw, fin)
    s_raw, stat = _reduce(ps.reshape(NW, E // D, D))
    mu2 = stat[0] / E
    var2 = stat[1] / E - mu2 * mu2
    k2 = gamma2[0] / jnp.sqrt(var2 + EPS)
    b2 = beta2[0] - mu2 * k2
    scal = jnp.stack([k2, b2])

    out = _final(s_raw, fing.reshape(E // D, D), scal)
    return out.reshape(N_AG, DEG)


# speedup vs baseline: 1.3264x; 1.3264x over previous
import functools

import jax
import jax.numpy as jnp
from jax import lax
from jax.experimental import pallas as pl
from jax.experimental.pallas import tpu as pltpu
from jax.experimental.pallas import tpu_sc as plsc

N_TASK = 5000
N_AG = 5000
D = 128
DEG = 64
E = N_AG * DEG
FIN_TASK_TYPE = 3
EPS = 1e-5
NEG_SLOPE = 0.01

NC, NS = 2, 16
NW = NC * NS
FPW = D // NW
CH = 1600
NCH = E // CH
AGC = CH // DEG
EFW = E // NW

_SC_MESH = plsc.VectorSubcoreMesh(
    core_axis_name="c", subcore_axis_name="s", num_cores=NC, num_subcores=NS)
_SC_PARAMS = pltpu.CompilerParams(needs_layout_passes=False)


def _proj_body(nt_ref, na_ref, ws_ref, wd_ref, t_ref, a_ref):
    t_ref[...] = jnp.dot(nt_ref[...], ws_ref[...],
                         preferred_element_type=jnp.float32)
    a_ref[...] = jnp.dot(na_ref[...], wd_ref[...],
                         preferred_element_type=jnp.float32)


def _proj(nf_t, nf_a, w1s, w1d):
    blk = 1000
    return pl.pallas_call(
        _proj_body,
        grid=(N_TASK // blk,),
        in_specs=[
            pl.BlockSpec((blk, D), lambda i: (i, 0)),
            pl.BlockSpec((blk, D), lambda i: (i, 0)),
            pl.BlockSpec((D, D), lambda i: (0, 0)),
            pl.BlockSpec((D, D), lambda i: (0, 0)),
        ],
        out_specs=[
            pl.BlockSpec((blk, D), lambda i: (i, 0)),
            pl.BlockSpec((blk, D), lambda i: (i, 0)),
        ],
        out_shape=[
            jax.ShapeDtypeStruct((N_TASK, D), jnp.float32),
            jax.ShapeDtypeStruct((N_AG, D), jnp.float32),
        ],
    )(nf_t, nf_a, w1s, w1d)


def _load_tables(tpT, apT, wid, tl, al):
    for c in range(FPW):
        pltpu.sync_copy(tpT.at[pl.ds((wid * FPW + c) * N_TASK, N_TASK)],
                        tl[c])
    pltpu.sync_copy(apT.at[pl.ds(wid * FPW * N_AG, FPW * N_AG)], al)


def _s1_body(tpT, apT, src, part, t0, t1, t2, t3, al_v, sb0, sb1, acc_v,
             sem0, sem1):
    wid = lax.axis_index("s") * NC + lax.axis_index("c")
    tl = [t0, t1, t2, t3]
    _load_tables(tpT, apT, wid, tl, al_v)

    def start_src(ch, sb, sem):
        pltpu.async_copy(src.at[pl.ds(ch * CH, CH)], sb, sem)

    def wait_src(ch, sb, sem):
        pltpu.make_async_copy(src.at[pl.ds(ch * CH, CH)], sb, sem).wait()

    zero = jnp.zeros((16,), jnp.float32)

    def process(ch, sb, accs):
        def agent_body(a, accs):
            s, q = accs
            agv = jnp.broadcast_to(ch * AGC + a, (16,)).astype(jnp.int32)
            asp = [plsc.load_gather(al_v, [agv + c * N_AG])
                   for c in range(FPW)]
            s = list(s)
            q = list(q)
            for g in range(DEG // 16):
                srcv = sb[pl.ds(a * DEG + g * 16, 16)]
                for c in range(FPW):
                    tg = plsc.load_gather(tl[c], [srcv])
                    h = tg + asp[c]
                    s[c] = s[c] + h
                    q[c] = q[c] + h * h
            return (tuple(s), tuple(q))

        return lax.fori_loop(0, AGC, agent_body, accs)

    start_src(0, sb0, sem0)
    init = (tuple(zero for _ in range(FPW)), tuple(zero for _ in range(FPW)))

    def body(p, accs):
        ch0, ch1 = 2 * p, 2 * p + 1
        start_src(ch1, sb1, sem1)
        wait_src(ch0, sb0, sem0)
        accs = process(ch0, sb0, accs)

        @pl.when(ch1 + 1 < NCH)
        def _():
            start_src(ch1 + 1, sb0, sem0)

        wait_src(ch1, sb1, sem1)
        return process(ch1, sb1, accs)

    fs, fq = lax.fori_loop(0, NCH // 2, body, init)
    for c in range(FPW):
        acc_v[pl.ds(c * 16, 16)] = fs[c]
        acc_v[pl.ds((FPW + c) * 16, 16)] = fq[c]
    pltpu.sync_copy(acc_v, part.at[wid])


@functools.partial(
    pl.kernel,
    out_type=jax.ShapeDtypeStruct((NW, 2 * FPW * 16), jnp.float32),
    mesh=_SC_MESH,
    compiler_params=_SC_PARAMS,
    scratch_types=[
        pltpu.VMEM((N_TASK,), jnp.float32),
        pltpu.VMEM((N_TASK,), jnp.float32),
        pltpu.VMEM((N_TASK,), jnp.float32),
        pltpu.VMEM((N_TASK,), jnp.float32),
        pltpu.VMEM((FPW * N_AG,), jnp.float32),
        pltpu.VMEM((CH,), jnp.int32),
        pltpu.VMEM((CH,), jnp.int32),
        pltpu.VMEM((2 * FPW * 16,), jnp.float32),
        pltpu.SemaphoreType.DMA,
        pltpu.SemaphoreType.DMA,
    ],
)
def _s1(tpT, apT, src, part, t0, t1, t2, t3, al_v, sb0, sb1, acc_v,
        sem0, sem1):
    _s1_body(tpT, apT, src, part, t0, t1, t2, t3, al_v, sb0, sb1, acc_v,
             sem0, sem1)


_AST = 5008


def _s2_body(tpT, apT, src, kbw, fin, ps, fing,
             t0, t1, t2, t3, al_v, sb0, sb1, pb0, pb1, kbw_v, fint_v,
             fsrc_v, finb_v, sem0, sem1, psem0, psem1, fsem):
    wid = lax.axis_index("s") * NC + lax.axis_index("c")
    tl = [t0, t1, t2, t3]
    for c in range(FPW):
        toff = pl.multiple_of((wid * FPW + c) * N_TASK, 8)
        pltpu.sync_copy(tpT.at[pl.ds(toff, N_TASK)],
                        tl[c].at[pl.ds(0, N_TASK)])
        aoff = pl.multiple_of((wid * FPW + c) * N_AG, 8)
        pltpu.sync_copy(apT.at[pl.ds(aoff, N_AG)],
                        al_v.at[pl.ds(c * _AST, N_AG)])
    pltpu.sync_copy(kbw, kbw_v)
    pltpu.sync_copy(fin, fint_v)

    widv = jnp.broadcast_to(wid * FPW, (16,)).astype(jnp.int32)
    k1sp = [plsc.load_gather(kbw_v, [widv + c]) for c in range(FPW)]
    b1sp = [plsc.load_gather(kbw_v, [widv + (D + c)]) for c in range(FPW)]
    w2sp = [plsc.load_gather(kbw_v, [widv + (2 * D + c)])
            for c in range(FPW)]

    def scale_t(i, cy):
        for c in range(FPW):
            tl[c][pl.ds(i * 16, 16)] = tl[c][pl.ds(i * 16, 16)] * k1sp[c]
        return cy

    lax.fori_loop(0, _AST // 16, scale_t, jnp.int32(0))

    def scale_a(i, cy):
        for c in range(FPW):
            al_v[pl.ds(c * _AST + i * 16, 16)] = (
                al_v[pl.ds(c * _AST + i * 16, 16)] * k1sp[c] + b1sp[c])
        return cy

    lax.fori_loop(0, _AST // 16, scale_a, jnp.int32(0))

    def start_src(ch, sb, sem):
        pltpu.async_copy(src.at[pl.ds(ch * CH, CH)], sb, sem)

    def wait_src(ch, sb, sem):
        pltpu.make_async_copy(src.at[pl.ds(ch * CH, CH)], sb, sem).wait()

    def start_ps(ch, pb, psem):
        off = pl.multiple_of(wid * E + ch * CH, 8)
        pltpu.async_copy(pb, ps.at[pl.ds(off, CH)], psem)

    def wait_ps(ch, pb, psem):
        off = pl.multiple_of(wid * E + ch * CH, 8)
        pltpu.make_async_copy(pb, ps.at[pl.ds(off, CH)], psem).wait()

    def process(ch, sb, pb):
        def agent_body(a):
            agv = jnp.broadcast_to(ch * AGC + a, (16,)).astype(jnp.int32)
            srcvs = [sb[pl.ds(a * DEG + g * 16, 16)]
                     for g in range(DEG // 16)]
            tgs = [[plsc.load_gather(tl[c], [srcvs[g]]) for c in range(FPW)]
                   for g in range(DEG // 16)]
            ca = [plsc.load_gather(al_v, [agv + c * _AST])
                  for c in range(FPW)]
            for g in range(DEG // 16):
                acc0 = None
                acc1 = None
                for c in range(FPW):
                    hn = tgs[g][c] + ca[c]
                    lr = jnp.maximum(hn, hn * NEG_SLOPE)
                    t = lr * w2sp[c]
                    if c % 2 == 0:
                        acc0 = t if acc0 is None else acc0 + t
                    else:
                        acc1 = t if acc1 is None else acc1 + t
                pb[pl.ds(a * DEG + g * 16, 16)] = acc0 + acc1
            return jnp.int32(0)

        lax.fori_loop(0, AGC, lambda a, cy: agent_body(a), jnp.int32(0))

    start_src(0, sb0, sem0)

    def body(p, carry):
        ch0, ch1 = 2 * p, 2 * p + 1
        pltpu.sync_copy(src.at[pl.ds(ch0 * CH, CH)], sb0)

        @pl.when(p > 0)
        def _():
            wait_ps(ch0 - 2, pb0, psem0)

        process(ch0, sb0, pb0)
        start_ps(ch0, pb0, psem0)

        pltpu.sync_copy(src.at[pl.ds(ch1 * CH, CH)], sb1)

        @pl.when(p > 0)
        def _():
            wait_ps(ch1 - 2, pb1, psem1)

        process(ch1, sb1, pb1)
        start_ps(ch1, pb1, psem1)
        return carry

    lax.fori_loop(0, NCH // 2, body, jnp.int32(0))

    for t in range(5):
        off = pl.multiple_of(wid * EFW + t * 2000, 8)
        pltpu.sync_copy(src.at[pl.ds(off, 2000)],
                        fsrc_v.at[pl.ds(t * 2000, 2000)])

    def fin_body(g, cy):
        srcv = fsrc_v[pl.ds(g * 16, 16)]
        finb_v[pl.ds(g * 16, 16)] = plsc.load_gather(fint_v, [srcv])
        return cy

    lax.fori_loop(0, EFW // 16, fin_body, jnp.int32(0))
    pltpu.sync_copy(finb_v, fing.at[pl.ds(wid * EFW, EFW)])

    wait_ps(NCH - 2, pb0, psem0)
    wait_ps(NCH - 1, pb1, psem1)


@functools.partial(
    pl.kernel,
    out_type=(
        jax.ShapeDtypeStruct((NW * E,), jnp.float32),
        jax.ShapeDtypeStruct((E,), jnp.float32),
    ),
    mesh=_SC_MESH,
    compiler_params=_SC_PARAMS,
    scratch_types=[
        pltpu.VMEM((_AST,), jnp.float32),
        pltpu.VMEM((_AST,), jnp.float32),
        pltpu.VMEM((_AST,), jnp.float32),
        pltpu.VMEM((_AST,), jnp.float32),
        pltpu.VMEM((FPW * _AST,), jnp.float32),
        pltpu.VMEM((CH,), jnp.int32),
        pltpu.VMEM((CH,), jnp.int32),
        pltpu.VMEM((CH,), jnp.float32),
        pltpu.VMEM((CH,), jnp.float32),
        pltpu.VMEM((3 * D,), jnp.float32),
        pltpu.VMEM((N_TASK,), jnp.float32),
        pltpu.VMEM((EFW,), jnp.int32),
        pltpu.VMEM((EFW,), jnp.float32),
        pltpu.SemaphoreType.DMA,
        pltpu.SemaphoreType.DMA,
        pltpu.SemaphoreType.DMA,
        pltpu.SemaphoreType.DMA,
        pltpu.SemaphoreType.DMA,
    ],
)
def _s2(tpT, apT, src, kbw, fin, ps, fing,
        t0, t1, t2, t3, al_v, sb0, sb1, pb0, pb1, kbw_v, fint_v, fsrc_v,
        finb_v, sem0, sem1, psem0, psem1, fsem):
    _s2_body(tpT, apT, src, kbw, fin, ps, fing,
             t0, t1, t2, t3, al_v, sb0, sb1, pb0, pb1, kbw_v, fint_v,
             fsrc_v, finb_v, sem0, sem1, psem0, psem1, fsem)


_TB = 8


def _red_body(ps_ref, s_ref, stat_ref):
    i = pl.program_id(0)
    sblk = jnp.sum(ps_ref[...], axis=0)

    @pl.when(i == 0)
    def _():
        s_ref[...] = sblk

    @pl.when(i > 0)
    def _():
        s_ref[...] += sblk

    @pl.when(i == pl.num_programs(0) - 1)
    def _():
        s_all = s_ref[...]
        stat_ref[0] = jnp.sum(s_all)
        stat_ref[1] = jnp.sum(s_all * s_all)


def _reduce(ps3):
    rows = E // D
    return pl.pallas_call(
        _red_body,
        grid=(NW // _TB,),
        in_specs=[pl.BlockSpec((_TB, rows, D), lambda i: (i, 0, 0))],
        out_specs=[
            pl.BlockSpec((rows, D), lambda i: (0, 0)),
            pl.BlockSpec(memory_space=pltpu.SMEM),
        ],
        out_shape=[
            jax.ShapeDtypeStruct((rows, D), jnp.float32),
            jax.ShapeDtypeStruct((2,), jnp.float32),
        ],
    )(ps3)


def _fin_body(s_ref, f_ref, scal_ref, o_ref):
    k2 = scal_ref[0]
    b2 = scal_ref[1]
    o_ref[...] = jnp.where(f_ref[...] > 0.5, -jnp.inf,
                           s_ref[...] * k2 + b2)


def _final(s2, f2, scal):
    rows = E // D
    return pl.pallas_call(
        _fin_body,
        grid=(1,),
        in_specs=[
            pl.BlockSpec((rows, D), lambda i: (0, 0)),
            pl.BlockSpec((rows, D), lambda i: (0, 0)),
            pl.BlockSpec(memory_space=pltpu.SMEM),
        ],
        out_specs=pl.BlockSpec((rows, D), lambda i: (0, 0)),
        out_shape=jax.ShapeDtypeStruct((rows, D), jnp.float32),
    )(s2, f2, scal)


def kernel(nf, edge_index, node_type, W1, gamma1, beta1, W2, gamma2, beta2):
    src = edge_index[0].astype(jnp.int32)
    nf_t = nf[:N_TASK]
    nf_a = nf[N_TASK:]
    w1s = W1[:D]
    w1d = W1[D:]

    t_proj, a_proj = _proj(nf_t, nf_a, w1s, w1d)
    tpT = t_proj.T.reshape(-1)
    apT = a_proj.T.reshape(-1)

    part = _s1(tpT, apT, src)
    plane = part.reshape(NW, 2 * FPW, 16).sum(axis=-1)
    sums = plane[:, :FPW].reshape(D)
    sqs = plane[:, FPW:].reshape(D)
    mu1 = sums / E
    var1 = sqs / E - mu1 * mu1
    k1 = gamma1 / jnp.sqrt(var1 + EPS)
    b1 = beta1 - mu1 * k1

    kbw = jnp.concatenate([k1, b1, W2[:, 0]])
    fin = (node_type[:N_TASK] == FIN_TASK_TYPE).astype(jnp.float32)

    ps, fing = _s2(tpT, apT, src, kbw, fin)
    s_raw, stat = _reduce(ps.reshape(NW, E // D, D))
    mu2 = stat[0] / E
    var2 = stat[1] / E - mu2 * mu2
    k2 = gamma2[0] / jnp.sqrt(var2 + EPS)
    b2 = beta2[0] - mu2 * k2
    scal = jnp.stack([k2, b2])

    out = _final(s_raw, fing.reshape(E // D, D), scal)
    return out.reshape(N_AG, DEG)


# speedup vs baseline: 1.6204x; 1.2217x over previous
import functools

import jax
import jax.numpy as jnp
from jax import lax
from jax.experimental import pallas as pl
from jax.experimental.pallas import tpu as pltpu
from jax.experimental.pallas import tpu_sc as plsc

N_TASK = 5000
N_AG = 5000
D = 128
DEG = 64
E = N_AG * DEG
FIN_TASK_TYPE = 3
EPS = 1e-5
NEG_SLOPE = 0.01

NC, NS = 2, 16
NW = NC * NS
FPW = D // NW
CH = 6400
NCH = E // CH
AGC = CH // DEG
EFW = E // NW

_SC_MESH = plsc.VectorSubcoreMesh(
    core_axis_name="c", subcore_axis_name="s", num_cores=NC, num_subcores=NS)
_SC_PARAMS = pltpu.CompilerParams(needs_layout_passes=False)


def _proj_body(nt_ref, na_ref, ws_ref, wd_ref, t_ref, a_ref):
    t_ref[...] = jnp.dot(nt_ref[...], ws_ref[...],
                         preferred_element_type=jnp.float32)
    a_ref[...] = jnp.dot(na_ref[...], wd_ref[...],
                         preferred_element_type=jnp.float32)


def _proj(nf_t, nf_a, w1s, w1d):
    blk = 1000
    return pl.pallas_call(
        _proj_body,
        grid=(N_TASK // blk,),
        in_specs=[
            pl.BlockSpec((blk, D), lambda i: (i, 0)),
            pl.BlockSpec((blk, D), lambda i: (i, 0)),
            pl.BlockSpec((D, D), lambda i: (0, 0)),
            pl.BlockSpec((D, D), lambda i: (0, 0)),
        ],
        out_specs=[
            pl.BlockSpec((blk, D), lambda i: (i, 0)),
            pl.BlockSpec((blk, D), lambda i: (i, 0)),
        ],
        out_shape=[
            jax.ShapeDtypeStruct((N_TASK, D), jnp.float32),
            jax.ShapeDtypeStruct((N_AG, D), jnp.float32),
        ],
    )(nf_t, nf_a, w1s, w1d)


def _load_tables(tpT, apT, wid, tl, al):
    for c in range(FPW):
        pltpu.sync_copy(tpT.at[pl.ds((wid * FPW + c) * N_TASK, N_TASK)],
                        tl[c])
    pltpu.sync_copy(apT.at[pl.ds(wid * FPW * N_AG, FPW * N_AG)], al)


def _s1_body(tpT, apT, src, part, t0, t1, t2, t3, al_v, sb0, sb1, acc_v,
             sem0, sem1):
    wid = lax.axis_index("s") * NC + lax.axis_index("c")
    tl = [t0, t1, t2, t3]
    _load_tables(tpT, apT, wid, tl, al_v)

    def start_src(ch, sb, sem):
        pltpu.async_copy(src.at[pl.ds(ch * CH, CH)], sb, sem)

    def wait_src(ch, sb, sem):
        pltpu.make_async_copy(src.at[pl.ds(ch * CH, CH)], sb, sem).wait()

    zero = jnp.zeros((16,), jnp.float32)

    def process(ch, sb, accs):
        def agent_body(a, accs):
            s, q = accs
            agv = jnp.broadcast_to(ch * AGC + a, (16,)).astype(jnp.int32)
            asp = [plsc.load_gather(al_v, [agv + c * N_AG])
                   for c in range(FPW)]
            s = list(s)
            q = list(q)
            for g in range(DEG // 16):
                srcv = sb[pl.ds(a * DEG + g * 16, 16)]
                for c in range(FPW):
                    tg = plsc.load_gather(tl[c], [srcv])
                    h = tg + asp[c]
                    s[c] = s[c] + h
                    q[c] = q[c] + h * h
            return (tuple(s), tuple(q))

        return lax.fori_loop(0, AGC, agent_body, accs)

    start_src(0, sb0, sem0)
    init = (tuple(zero for _ in range(FPW)), tuple(zero for _ in range(FPW)))

    def body(p, accs):
        ch0, ch1 = 2 * p, 2 * p + 1
        start_src(ch1, sb1, sem1)
        wait_src(ch0, sb0, sem0)
        accs = process(ch0, sb0, accs)

        @pl.when(ch1 + 1 < NCH)
        def _():
            start_src(ch1 + 1, sb0, sem0)

        wait_src(ch1, sb1, sem1)
        return process(ch1, sb1, accs)

    fs, fq = lax.fori_loop(0, NCH // 2, body, init)
    for c in range(FPW):
        acc_v[pl.ds(c * 16, 16)] = fs[c]
        acc_v[pl.ds((FPW + c) * 16, 16)] = fq[c]
    pltpu.sync_copy(acc_v, part.at[wid])


@functools.partial(
    pl.kernel,
    out_type=jax.ShapeDtypeStruct((NW, 2 * FPW * 16), jnp.float32),
    mesh=_SC_MESH,
    compiler_params=_SC_PARAMS,
    scratch_types=[
        pltpu.VMEM((N_TASK,), jnp.float32),
        pltpu.VMEM((N_TASK,), jnp.float32),
        pltpu.VMEM((N_TASK,), jnp.float32),
        pltpu.VMEM((N_TASK,), jnp.float32),
        pltpu.VMEM((FPW * N_AG,), jnp.float32),
        pltpu.VMEM((CH,), jnp.int32),
        pltpu.VMEM((CH,), jnp.int32),
        pltpu.VMEM((2 * FPW * 16,), jnp.float32),
        pltpu.SemaphoreType.DMA,
        pltpu.SemaphoreType.DMA,
    ],
)
def _s1(tpT, apT, src, part, t0, t1, t2, t3, al_v, sb0, sb1, acc_v,
        sem0, sem1):
    _s1_body(tpT, apT, src, part, t0, t1, t2, t3, al_v, sb0, sb1, acc_v,
             sem0, sem1)


_AST = 5008


def _s2_body(tpT, apT, src, kbw, fin, ps, fing,
             t0, t1, t2, t3, al_v, sb0, sb1, pb0, pb1, kbw_v, fint_v,
             fsrc_v, finb_v, sem0, sem1, psem0, psem1, fsem):
    wid = lax.axis_index("s") * NC + lax.axis_index("c")
    tl = [t0, t1, t2, t3]
    for c in range(FPW):
        toff = pl.multiple_of((wid * FPW + c) * N_TASK, 8)
        pltpu.sync_copy(tpT.at[pl.ds(toff, N_TASK)],
                        tl[c].at[pl.ds(0, N_TASK)])
        aoff = pl.multiple_of((wid * FPW + c) * N_AG, 8)
        pltpu.sync_copy(apT.at[pl.ds(aoff, N_AG)],
                        al_v.at[pl.ds(c * _AST, N_AG)])
    pltpu.sync_copy(kbw, kbw_v)
    pltpu.sync_copy(fin, fint_v)

    widv = jnp.broadcast_to(wid * FPW, (16,)).astype(jnp.int32)
    k1sp = [plsc.load_gather(kbw_v, [widv + c]) for c in range(FPW)]
    b1sp = [plsc.load_gather(kbw_v, [widv + (D + c)]) for c in range(FPW)]
    w2sp = [plsc.load_gather(kbw_v, [widv + (2 * D + c)])
            for c in range(FPW)]

    def scale_t(i, cy):
        for c in range(FPW):
            tl[c][pl.ds(i * 16, 16)] = tl[c][pl.ds(i * 16, 16)] * k1sp[c]
        return cy

    lax.fori_loop(0, _AST // 16, scale_t, jnp.int32(0))

    def scale_a(i, cy):
        for c in range(FPW):
            al_v[pl.ds(c * _AST + i * 16, 16)] = (
                al_v[pl.ds(c * _AST + i * 16, 16)] * k1sp[c] + b1sp[c])
        return cy

    lax.fori_loop(0, _AST // 16, scale_a, jnp.int32(0))

    def start_src(ch, sb, sem):
        pltpu.async_copy(src.at[pl.ds(ch * CH, CH)], sb, sem)

    def wait_src(ch, sb, sem):
        pltpu.make_async_copy(src.at[pl.ds(ch * CH, CH)], sb, sem).wait()

    def start_ps(ch, pb, psem):
        off = pl.multiple_of(wid * E + ch * CH, 8)
        pltpu.async_copy(pb, ps.at[pl.ds(off, CH)], psem)

    def wait_ps(ch, pb, psem):
        off = pl.multiple_of(wid * E + ch * CH, 8)
        pltpu.make_async_copy(pb, ps.at[pl.ds(off, CH)], psem).wait()

    def process(ch, sb, pb):
        def agent_body(a):
            agv = jnp.broadcast_to(ch * AGC + a, (16,)).astype(jnp.int32)
            srcvs = [sb[pl.ds(a * DEG + g * 16, 16)]
                     for g in range(DEG // 16)]
            tgs = [[plsc.load_gather(tl[c], [srcvs[g]]) for c in range(FPW)]
                   for g in range(DEG // 16)]
            ca = [plsc.load_gather(al_v, [agv + c * _AST])
                  for c in range(FPW)]
            for g in range(DEG // 16):
                acc0 = None
                acc1 = None
                for c in range(FPW):
                    hn = tgs[g][c] + ca[c]
                    lr = jnp.maximum(hn, hn * NEG_SLOPE)
                    t = lr * w2sp[c]
                    if c % 2 == 0:
                        acc0 = t if acc0 is None else acc0 + t
                    else:
                        acc1 = t if acc1 is None else acc1 + t
                pb[pl.ds(a * DEG + g * 16, 16)] = acc0 + acc1
            return jnp.int32(0)

        lax.fori_loop(0, AGC, lambda a, cy: agent_body(a), jnp.int32(0))

    start_src(0, sb0, sem0)

    def body(p, carry):
        ch0, ch1 = 2 * p, 2 * p + 1
        pltpu.sync_copy(src.at[pl.ds(ch0 * CH, CH)], sb0)

        @pl.when(p > 0)
        def _():
            wait_ps(ch0 - 2, pb0, psem0)

        process(ch0, sb0, pb0)
        start_ps(ch0, pb0, psem0)

        pltpu.sync_copy(src.at[pl.ds(ch1 * CH, CH)], sb1)

        @pl.when(p > 0)
        def _():
            wait_ps(ch1 - 2, pb1, psem1)

        process(ch1, sb1, pb1)
        start_ps(ch1, pb1, psem1)
        return carry

    lax.fori_loop(0, NCH // 2, body, jnp.int32(0))

    for t in range(5):
        off = pl.multiple_of(wid * EFW + t * 2000, 8)
        pltpu.sync_copy(src.at[pl.ds(off, 2000)],
                        fsrc_v.at[pl.ds(t * 2000, 2000)])

    def fin_body(g, cy):
        srcv = fsrc_v[pl.ds(g * 16, 16)]
        finb_v[pl.ds(g * 16, 16)] = plsc.load_gather(fint_v, [srcv])
        return cy

    lax.fori_loop(0, EFW // 16, fin_body, jnp.int32(0))
    pltpu.sync_copy(finb_v, fing.at[pl.ds(wid * EFW, EFW)])

    wait_ps(NCH - 2, pb0, psem0)
    wait_ps(NCH - 1, pb1, psem1)


@functools.partial(
    pl.kernel,
    out_type=(
        jax.ShapeDtypeStruct((NW * E,), jnp.float32),
        jax.ShapeDtypeStruct((E,), jnp.float32),
    ),
    mesh=_SC_MESH,
    compiler_params=_SC_PARAMS,
    scratch_types=[
        pltpu.VMEM((_AST,), jnp.float32),
        pltpu.VMEM((_AST,), jnp.float32),
        pltpu.VMEM((_AST,), jnp.float32),
        pltpu.VMEM((_AST,), jnp.float32),
        pltpu.VMEM((FPW * _AST,), jnp.float32),
        pltpu.VMEM((CH,), jnp.int32),
        pltpu.VMEM((CH,), jnp.int32),
        pltpu.VMEM((CH,), jnp.float32),
        pltpu.VMEM((CH,), jnp.float32),
        pltpu.VMEM((3 * D,), jnp.float32),
        pltpu.VMEM((N_TASK,), jnp.float32),
        pltpu.VMEM((EFW,), jnp.int32),
        pltpu.VMEM((EFW,), jnp.float32),
        pltpu.SemaphoreType.DMA,
        pltpu.SemaphoreType.DMA,
        pltpu.SemaphoreType.DMA,
        pltpu.SemaphoreType.DMA,
        pltpu.SemaphoreType.DMA,
    ],
)
def _s2(tpT, apT, src, kbw, fin, ps, fing,
        t0, t1, t2, t3, al_v, sb0, sb1, pb0, pb1, kbw_v, fint_v, fsrc_v,
        finb_v, sem0, sem1, psem0, psem1, fsem):
    _s2_body(tpT, apT, src, kbw, fin, ps, fing,
             t0, t1, t2, t3, al_v, sb0, sb1, pb0, pb1, kbw_v, fint_v,
             fsrc_v, finb_v, sem0, sem1, psem0, psem1, fsem)


_TB = 8


def _red_body(ps_ref, s_ref, stat_ref):
    i = pl.program_id(0)
    sblk = jnp.sum(ps_ref[...], axis=0)

    @pl.when(i == 0)
    def _():
        s_ref[...] = sblk

    @pl.when(i > 0)
    def _():
        s_ref[...] += sblk

    @pl.when(i == pl.num_programs(0) - 1)
    def _():
        s_all = s_ref[...]
        stat_ref[0] = jnp.sum(s_all)
        stat_ref[1] = jnp.sum(s_all * s_all)


def _reduce(ps2):
    rows = E // D
    return pl.pallas_call(
        _red_body,
        grid=(NW // _TB,),
        in_specs=[pl.BlockSpec((_TB, rows, D), lambda i: (i, 0, 0))],
        out_specs=[
            pl.BlockSpec((rows, D), lambda i: (0, 0)),
            pl.BlockSpec(memory_space=pltpu.SMEM),
        ],
        out_shape=[
            jax.ShapeDtypeStruct((rows, D), jnp.float32),
            jax.ShapeDtypeStruct((2,), jnp.float32),
        ],
    )(ps2)


def _fin_body(s_ref, f_ref, scal_ref, o_ref):
    k2 = scal_ref[0]
    b2 = scal_ref[1]
    o_ref[...] = jnp.where(f_ref[...] > 0.5, -jnp.inf,
                           s_ref[...] * k2 + b2)


def _final(s2, f2, scal):
    rows = E // D
    return pl.pallas_call(
        _fin_body,
        grid=(1,),
        in_specs=[
            pl.BlockSpec((rows, D), lambda i: (0, 0)),
            pl.BlockSpec((rows, D), lambda i: (0, 0)),
            pl.BlockSpec(memory_space=pltpu.SMEM),
        ],
        out_specs=pl.BlockSpec((rows, D), lambda i: (0, 0)),
        out_shape=jax.ShapeDtypeStruct((rows, D), jnp.float32),
    )(s2, f2, scal)


def kernel(nf, edge_index, node_type, W1, gamma1, beta1, W2, gamma2, beta2):
    src = edge_index[0].astype(jnp.int32)
    nf_t = nf[:N_TASK]
    nf_a = nf[N_TASK:]
    w1s = W1[:D]
    w1d = W1[D:]

    t_proj, a_proj = _proj(nf_t, nf_a, w1s, w1d)
    tpT = t_proj.T.reshape(-1)
    apT = a_proj.T.reshape(-1)

    part = _s1(tpT, apT, src)
    plane = part.reshape(NW, 2 * FPW, 16).sum(axis=-1)
    sums = plane[:, :FPW].reshape(D)
    sqs = plane[:, FPW:].reshape(D)
    mu1 = sums / E
    var1 = sqs / E - mu1 * mu1
    k1 = gamma1 / jnp.sqrt(var1 + EPS)
    b1 = beta1 - mu1 * k1

    kbw = jnp.concatenate([k1, b1, W2[:, 0]])
    fin = (node_type[:N_TASK] == FIN_TASK_TYPE).astype(jnp.float32)

    ps, fing = _s2(tpT, apT, src, kbw, fin)
    s_raw, stat = _reduce(ps.reshape(NW, E // D, D))
    mu2 = stat[0] / E
    var2 = stat[1] / E - mu2 * mu2
    k2 = gamma2[0] / jnp.sqrt(var2 + EPS)
    b2 = beta2[0] - mu2 * k2
    scal = jnp.stack([k2, b2])

    out = _final(s_raw, fing.reshape(E // D, D), scal)
    return out.reshape(N_AG, DEG)
